# Initial kernel scaffold; baseline (speedup 1.0000x reference)
import jax
import jax.numpy as jnp
from jax.experimental import pallas as pl


def kernel(inputs, params):
    raise NotImplementedError("write your pallas kernel here")


# speedup vs baseline: 4.7181x; 4.7181x over previous
import jax
import jax.numpy as jnp
from jax.experimental import pallas as pl


def _body(p_ref, o_ref):
    o_ref[...] = jnp.broadcast_to(p_ref[...][None], o_ref.shape)


def kernel(inputs, params):
    n, t, d = inputs.shape
    bt = 256
    return pl.pallas_call(
        _body,
        grid=(t // bt,),
        in_specs=[pl.BlockSpec((bt, d), lambda i: (i, 0))],
        out_specs=pl.BlockSpec((n, bt, d), lambda i: (0, i, 0)),
        out_shape=jax.ShapeDtypeStruct((n, t, d), params.dtype),
    )(params)


# speedup vs baseline: 5.2318x; 1.1089x over previous
import jax
import jax.numpy as jnp
from jax.experimental import pallas as pl


def _body(p_ref, o_ref):
    o_ref[...] = jnp.broadcast_to(p_ref[...][None], o_ref.shape)


def kernel(inputs, params):
    n, t, d = inputs.shape
    bt = 512
    return pl.pallas_call(
        _body,
        grid=(t // bt,),
        in_specs=[pl.BlockSpec((bt, d), lambda i: (i, 0))],
        out_specs=pl.BlockSpec((n, bt, d), lambda i: (0, i, 0)),
        out_shape=jax.ShapeDtypeStruct((n, t, d), params.dtype),
    )(params)
